# Initial kernel scaffold; baseline (speedup 1.0000x reference)
import functools

import jax
import jax.numpy as jnp
from jax import lax
from jax.experimental import pallas as pl
from jax.experimental.pallas import tpu as pltpu
from jax.experimental.pallas import tpu_sc as plsc

N = 10000
D = 128
H = 128
C = 40
EPS = 1e-5

NC = 2
NS = 16
NW = NC * NS
CHUNK = 128
N_PAD = 10240
ROWS_PER_TILE = N_PAD // NS


def _sc_mesh():
    return plsc.VectorSubcoreMesh(core_axis_name="c", subcore_axis_name="s")


def _make_sc_agg(n_chunks, width, with_count):
    out_type = [jax.ShapeDtypeStruct((NC, N_PAD, width), jnp.float32)]
    scratch = [
        pltpu.VMEM((n_chunks, CHUNK), jnp.int32),
        pltpu.VMEM((n_chunks, CHUNK), jnp.int32),
        pltpu.VMEM((CHUNK, width), jnp.float32),
        pltpu.VMEM_SHARED((N_PAD, width), jnp.float32),
        pltpu.SemaphoreType.DMA,
    ]
    if with_count:
        out_type.append(jax.ShapeDtypeStruct((NC, N_PAD), jnp.float32))
        scratch.append(pltpu.VMEM((N_PAD,), jnp.float32))
        scratch.append(pltpu.VMEM_SHARED((N_PAD,), jnp.float32))

    def body(p_hbm, src_hbm, dst_hbm, *refs):
        if with_count:
            acc_out, cnt_out, src_v, dst_v, rows_v, acc_sh, sem, cnt_v, cnt_sh = refs
        else:
            acc_out, src_v, dst_v, rows_v, acc_sh, sem = refs
            cnt_out = cnt_v = cnt_sh = None
        c = lax.axis_index("c")
        s = lax.axis_index("s")
        wid = c * NS + s

        zeros16 = jnp.zeros((16,), jnp.float32)

        @pl.loop(0, CHUNK)
        def _(r):
            for k in range(width // 16):
                rows_v[r, pl.ds(k * 16, 16)] = zeros16

        row0 = s * ROWS_PER_TILE

        @pl.loop(0, ROWS_PER_TILE // CHUNK)
        def _(b):
            pltpu.sync_copy(rows_v, acc_sh.at[pl.ds(row0 + b * CHUNK, CHUNK)])

        if with_count:
            @pl.loop(0, N_PAD // 16)
            def _(i):
                cnt_v[pl.ds(i * 16, 16)] = zeros16

            @pl.when(s == 0)
            def _():
                pltpu.sync_copy(cnt_v, cnt_sh)

        pltpu.sync_copy(src_hbm.at[wid], src_v)
        pltpu.sync_copy(dst_hbm.at[wid], dst_v)

        plsc.subcore_barrier()

        ones16 = jnp.ones((16,), jnp.float32)

        @pl.loop(0, n_chunks)
        def _(j):
            pltpu.async_copy(p_hbm.at[src_v.at[j]], rows_v, sem).wait()
            pltpu.sync_copy(rows_v, acc_sh.at[dst_v.at[j]], add=True)
            if with_count:
                for k in range(CHUNK // 16):
                    idx = dst_v[j, pl.ds(k * 16, 16)]
                    plsc.addupdate_scatter(cnt_v, [idx], ones16)

        plsc.subcore_barrier()

        if with_count:
            pltpu.sync_copy(cnt_v, cnt_sh, add=True)
            plsc.subcore_barrier()

        rows = pl.ds(row0, ROWS_PER_TILE)
        pltpu.sync_copy(acc_sh.at[rows], acc_out.at[c, rows])
        if with_count:
            @pl.when(s == 0)
            def _():
                pltpu.sync_copy(cnt_sh, cnt_out.at[c])

    return pl.kernel(
        body,
        out_type=tuple(out_type) if with_count else out_type[0],
        mesh=_sc_mesh(),
        scratch_types=scratch,
    )


def _tc_first(x, wl):
    def body(x_ref, w_ref, o_ref):
        o_ref[...] = jnp.dot(x_ref[...], w_ref[...],
                             preferred_element_type=jnp.float32)

    return pl.pallas_call(
        body,
        out_shape=jax.ShapeDtypeStruct((N, wl.shape[1]), jnp.float32),
    )(x, wl)


def _tc_mid(agg, cnt, xin, wr, b, g, be, wl_next):
    wn = wl_next.shape[1]

    def body(agg_ref, cnt_ref, x_ref, wr_ref, b_ref, g_ref, be_ref,
             wln_ref, h_ref, p_ref):
        cnt_sum = cnt_ref[0, :N] + cnt_ref[1, :N]
        denom = jnp.maximum(cnt_sum, 1.0)
        agg_sum = agg_ref[0, :N, :] + agg_ref[1, :N, :]
        mean = agg_sum / denom[:, None]
        xv = x_ref[...]
        hpre = mean + jnp.dot(xv, wr_ref[...],
                              preferred_element_type=jnp.float32) + b_ref[...]
        mu = jnp.mean(hpre, axis=0)
        var = jnp.mean((hpre - mu) ** 2, axis=0)
        hn = (hpre - mu) / jnp.sqrt(var + EPS) * g_ref[...] + be_ref[...]
        h = jnp.maximum(hn, 0.0)
        h_ref[...] = h
        p_ref[...] = jnp.dot(h, wln_ref[...],
                             preferred_element_type=jnp.float32)

    return pl.pallas_call(
        body,
        out_shape=(
            jax.ShapeDtypeStruct((N, H), jnp.float32),
            jax.ShapeDtypeStruct((N, wn), jnp.float32),
        ),
    )(agg, cnt, xin, wr, b, g, be, wl_next)


def _tc_final(agg, cnt, xin, wr, b):
    def body(agg_ref, cnt_ref, x_ref, wr_ref, b_ref, o_ref):
        cnt_sum = cnt_ref[0, :N] + cnt_ref[1, :N]
        denom = jnp.maximum(cnt_sum, 1.0)
        agg_sum = agg_ref[0, :N, :C] + agg_ref[1, :N, :C]
        mean = agg_sum / denom[:, None]
        o = mean + jnp.dot(x_ref[...], wr_ref[...],
                           preferred_element_type=jnp.float32) + b_ref[...]
        m = jnp.max(o, axis=1, keepdims=True)
        z = o - m
        lse = jnp.log(jnp.sum(jnp.exp(z), axis=1, keepdims=True))
        o_ref[...] = z - lse

    return pl.pallas_call(
        body,
        out_shape=jax.ShapeDtypeStruct((N, C), jnp.float32),
    )(agg, cnt, xin, wr, b)


def kernel(x, edge_index, Wl1, Wr1, b1, g1, be1, Wl2, Wr2, b2, g2, be2,
           Wl3, Wr3, b3):
    e = edge_index.shape[1]
    n_chunks = -(-e // (NW * CHUNK))
    e_pad = NW * n_chunks * CHUNK

    src = edge_index[0].astype(jnp.int32)
    dst = edge_index[1].astype(jnp.int32)
    pad = e_pad - e
    if pad:
        src = jnp.concatenate([src, jnp.zeros((pad,), jnp.int32)])
        dst = jnp.concatenate([dst, jnp.full((pad,), N, jnp.int32)])
    src3 = src.reshape(NW, n_chunks, CHUNK)
    dst3 = dst.reshape(NW, n_chunks, CHUNK)

    wc = 48
    wl3p = jnp.concatenate(
        [Wl3, jnp.zeros((H, wc - C), jnp.float32)], axis=1)

    agg128 = _make_sc_agg(n_chunks, 128, True)
    agg128nc = _make_sc_agg(n_chunks, 128, False)
    agg48 = _make_sc_agg(n_chunks, wc, False)

    p1 = _tc_first(x, Wl1)
    a1, cnt = agg128(p1, src3, dst3)
    h1, p2 = _tc_mid(a1, cnt, x, Wr1, b1, g1, be1, Wl2)
    a2 = agg128nc(p2, src3, dst3)
    h2, p3 = _tc_mid(a2, cnt, h1, Wr2, b2, g2, be2, wl3p)
    a3 = agg48(p3, src3, dst3)
    return _tc_final(a3, cnt, h2, Wr3, b3)


# speedup vs baseline: 5.4603x; 5.4603x over previous
import jax
import jax.numpy as jnp
from jax import lax
from jax.experimental import pallas as pl
from jax.experimental.pallas import tpu as pltpu
from jax.experimental.pallas import tpu_sc as plsc

N = 10000
D = 128
H = 128
C = 40
EPS = 1e-5

NC = 2
NS = 16
NW = NC * NS
CHUNK = 128
N_PAD = 10240
ROWS_PER_TILE = N_PAD // NS
W1 = 144
W3 = 48


def _make_sc_agg(n_chunks, width):
    def body(p_hbm, src_hbm, dst_hbm, acc_out, src_v, dst_v, rows_v,
             acc_sh, sem):
        c = lax.axis_index("c")
        s = lax.axis_index("s")
        wid = c * NS + s

        zeros16 = jnp.zeros((16,), jnp.float32)

        @pl.loop(0, CHUNK)
        def _(r):
            for k in range(width // 16):
                rows_v[r, pl.ds(k * 16, 16)] = zeros16

        row0 = s * ROWS_PER_TILE

        @pl.loop(0, ROWS_PER_TILE // CHUNK)
        def _(b):
            pltpu.sync_copy(rows_v, acc_sh.at[pl.ds(row0 + b * CHUNK, CHUNK)])

        pltpu.sync_copy(src_hbm.at[wid], src_v)
        pltpu.sync_copy(dst_hbm.at[wid], dst_v)

        plsc.subcore_barrier()

        @pl.loop(0, n_chunks)
        def _(j):
            pltpu.async_copy(p_hbm.at[src_v.at[j]], rows_v, sem).wait()
            pltpu.sync_copy(rows_v, acc_sh.at[dst_v.at[j]], add=True)

        plsc.subcore_barrier()

        rows = pl.ds(row0, ROWS_PER_TILE)
        pltpu.sync_copy(acc_sh.at[rows], acc_out.at[c, rows])

    return pl.kernel(
        body,
        out_type=jax.ShapeDtypeStruct((NC, N_PAD, width), jnp.float32),
        mesh=plsc.VectorSubcoreMesh(core_axis_name="c", subcore_axis_name="s"),
        compiler_params=pltpu.CompilerParams(use_tc_tiling_on_sc=False),
        scratch_types=[
            pltpu.VMEM((n_chunks, CHUNK), jnp.int32),
            pltpu.VMEM((n_chunks, CHUNK), jnp.int32),
            pltpu.VMEM((CHUNK, width), jnp.float32),
            pltpu.VMEM_SHARED((N_PAD, width), jnp.float32),
            pltpu.SemaphoreType.DMA,
        ],
    )


def _tc_first(x, wl):
    def body(x_ref, w_ref, o_ref):
        o_ref[:, :H] = jnp.dot(x_ref[...], w_ref[...],
                               preferred_element_type=jnp.float32)
        col = lax.broadcasted_iota(jnp.int32, (N, W1 - H), 1)
        o_ref[:, H:] = jnp.where(col == 0, 1.0, 0.0)

    return pl.pallas_call(
        body,
        out_shape=jax.ShapeDtypeStruct((N, W1), jnp.float32),
    )(x, wl)


def _tc_mid1(agg, x, wr, b, g, be, wl_next):
    def body(agg_ref, x_ref, wr_ref, b_ref, g_ref, be_ref, wln_ref,
             h_ref, p_ref, den_ref):
        asum = agg_ref[0, :N, :] + agg_ref[1, :N, :]
        denom = jnp.maximum(asum[:, H:H + 1], 1.0)
        den_ref[...] = denom
        mean = asum[:, :H] / denom
        hpre = mean + jnp.dot(x_ref[...], wr_ref[...],
                              preferred_element_type=jnp.float32) + b_ref[...]
        mu = jnp.mean(hpre, axis=0)
        var = jnp.mean((hpre - mu) ** 2, axis=0)
        h = jnp.maximum((hpre - mu) / jnp.sqrt(var + EPS) * g_ref[...]
                        + be_ref[...], 0.0)
        h_ref[...] = h
        p_ref[...] = jnp.dot(h, wln_ref[...],
                             preferred_element_type=jnp.float32)

    return pl.pallas_call(
        body,
        out_shape=(
            jax.ShapeDtypeStruct((N, H), jnp.float32),
            jax.ShapeDtypeStruct((N, H), jnp.float32),
            jax.ShapeDtypeStruct((N, 1), jnp.float32),
        ),
    )(agg, x, wr, b, g, be, wl_next)


def _tc_mid2(agg, den, xin, wr, b, g, be, wl_next):
    def body(agg_ref, den_ref, x_ref, wr_ref, b_ref, g_ref, be_ref,
             wln_ref, h_ref, p_ref):
        asum = agg_ref[0, :N, :] + agg_ref[1, :N, :]
        mean = asum / den_ref[...]
        hpre = mean + jnp.dot(x_ref[...], wr_ref[...],
                              preferred_element_type=jnp.float32) + b_ref[...]
        mu = jnp.mean(hpre, axis=0)
        var = jnp.mean((hpre - mu) ** 2, axis=0)
        h = jnp.maximum((hpre - mu) / jnp.sqrt(var + EPS) * g_ref[...]
                        + be_ref[...], 0.0)
        h_ref[...] = h
        p_ref[...] = jnp.dot(h, wln_ref[...],
                             preferred_element_type=jnp.float32)

    return pl.pallas_call(
        body,
        out_shape=(
            jax.ShapeDtypeStruct((N, H), jnp.float32),
            jax.ShapeDtypeStruct((N, wl_next.shape[1]), jnp.float32),
        ),
    )(agg, den, xin, wr, b, g, be, wl_next)


def _tc_final(agg, den, xin, wr, b):
    def body(agg_ref, den_ref, x_ref, wr_ref, b_ref, o_ref):
        asum = agg_ref[0, :N, :C] + agg_ref[1, :N, :C]
        mean = asum / den_ref[...]
        o = mean + jnp.dot(x_ref[...], wr_ref[...],
                           preferred_element_type=jnp.float32) + b_ref[...]
        m = jnp.max(o, axis=1, keepdims=True)
        z = o - m
        lse = jnp.log(jnp.sum(jnp.exp(z), axis=1, keepdims=True))
        o_ref[...] = z - lse

    return pl.pallas_call(
        body,
        out_shape=jax.ShapeDtypeStruct((N, C), jnp.float32),
    )(agg, den, xin, wr, b)


def kernel(x, edge_index, Wl1, Wr1, b1, g1, be1, Wl2, Wr2, b2, g2, be2,
           Wl3, Wr3, b3):
    e = edge_index.shape[1]
    n_chunks = -(-e // (NW * CHUNK))
    e_pad = NW * n_chunks * CHUNK

    src = edge_index[0].astype(jnp.int32)
    dst = edge_index[1].astype(jnp.int32)
    pad = e_pad - e
    if pad:
        src = jnp.concatenate([src, jnp.zeros((pad,), jnp.int32)])
        dst = jnp.concatenate([dst, jnp.full((pad,), N, jnp.int32)])
    src3 = src.reshape(NW, n_chunks, CHUNK)
    dst3 = dst.reshape(NW, n_chunks, CHUNK)

    wl3p = jnp.concatenate(
        [Wl3, jnp.zeros((H, W3 - C), jnp.float32)], axis=1)

    p1 = _tc_first(x, Wl1)
    a1 = _make_sc_agg(n_chunks, W1)(p1, src3, dst3)
    h1, p2, den = _tc_mid1(a1, x, Wr1, b1, g1, be1, Wl2)
    a2 = _make_sc_agg(n_chunks, H)(p2, src3, dst3)
    h2, p3 = _tc_mid2(a2, den, h1, Wr2, b2, g2, be2, wl3p)
    a3 = _make_sc_agg(n_chunks, W3)(p3, src3, dst3)
    return _tc_final(a3, den, h2, Wr3, b3)
